# baseline (speedup 1.0000x reference)
import jax
import jax.numpy as jnp
from jax.experimental import pallas as pl
from jax.experimental.pallas import tpu as pltpu

_LANE = 128


def _rowdot2_kernel(b_ref, xa_ref, xb_ref, w_ref, oa_ref, ob_ref):
    bias = b_ref[0, 0]
    oa_ref[...] = jnp.sum(xa_ref[...] * w_ref[...], axis=2) + bias
    ob_ref[...] = jnp.sum(xb_ref[...] * w_ref[...], axis=2) + bias


def _pick_block(n, candidates):
    for c in candidates:
        if n % c == 0:
            return c
    return 1


def kernel(x, wt_padded, b_padded):
    B, F = x.shape
    dtype = x.dtype

    n_rows = B
    pad = (-n_rows) % (2 * _LANE)
    if pad:
        x = jnp.pad(x, ((0, pad), (0, 0)))
        B = x.shape[0]

    s_total = B // _LANE
    half = s_total // 2
    x3 = x.reshape(s_total, _LANE, F)
    w3 = wt_padded[:, :1].reshape(1, 1, F)
    b11 = b_padded[:1, :1]

    s_blk = _pick_block(half, (32, 16, 8, 4, 2, 1))
    n_steps = half // s_blk
    grid = (n_steps,)

    oa, ob = pl.pallas_call(
        _rowdot2_kernel,
        out_shape=(
            jax.ShapeDtypeStruct((half, _LANE), dtype),
            jax.ShapeDtypeStruct((half, _LANE), dtype),
        ),
        grid_spec=pl.GridSpec(
            grid=grid,
            in_specs=[
                pl.BlockSpec(memory_space=pltpu.SMEM),
                pl.BlockSpec((s_blk, _LANE, F), lambda i: (i, 0, 0)),
                pl.BlockSpec((s_blk, _LANE, F),
                             lambda i, _n=n_steps: (i + _n, 0, 0)),
                pl.BlockSpec((1, 1, F), lambda i: (0, 0, 0)),
            ],
            out_specs=(
                pl.BlockSpec((s_blk, _LANE), lambda i: (i, 0)),
                pl.BlockSpec((s_blk, _LANE), lambda i: (i, 0)),
            ),
        ),
        compiler_params=pltpu.CompilerParams(
            dimension_semantics=("arbitrary",),
        ),
        cost_estimate=pl.CostEstimate(
            flops=2 * B * F,
            transcendentals=0,
            bytes_accessed=B * F * 4 + F * 4 + B * 4,
        ),
    )(b11, x3, x3, w3)

    out = jnp.concatenate([oa, ob], axis=0)
    return out.reshape(B, 1)[:n_rows]


# speedup vs baseline: 1.1302x; 1.1302x over previous
import jax
import jax.numpy as jnp
from jax.experimental import pallas as pl
from jax.experimental.pallas import tpu as pltpu

_LANE = 128


def _rowdot_kernel(b_ref, x_ref, w_ref, o_ref):
    w_lane = w_ref[...][:, 0].reshape(1, 1, w_ref.shape[0])
    z = x_ref[...] * w_lane
    o_ref[...] = jnp.sum(z, axis=2) + b_ref[0, 0]


def _pick_block(n, candidates):
    for c in candidates:
        if n % c == 0:
            return c
    return 1


def kernel(x, wt_padded, b_padded):
    B, F = x.shape
    dtype = x.dtype

    n_rows = B
    pad = (-n_rows) % _LANE
    if pad:
        x = jnp.pad(x, ((0, pad), (0, 0)))
        B = x.shape[0]

    s_total = B // _LANE
    x3 = x.reshape(s_total, _LANE, F)
    n_pad = wt_padded.shape[1]

    s_blk = _pick_block(s_total, (64, 32, 16, 8, 4, 2, 1))
    grid = (s_total // s_blk,)

    out = pl.pallas_call(
        _rowdot_kernel,
        out_shape=jax.ShapeDtypeStruct((s_total, _LANE), dtype),
        grid_spec=pl.GridSpec(
            grid=grid,
            in_specs=[
                pl.BlockSpec(memory_space=pltpu.SMEM),
                pl.BlockSpec((s_blk, _LANE, F), lambda i: (i, 0, 0)),
                pl.BlockSpec((F, n_pad), lambda i: (0, 0)),
            ],
            out_specs=pl.BlockSpec((s_blk, _LANE), lambda i: (i, 0)),
        ),
        compiler_params=pltpu.CompilerParams(
            dimension_semantics=("arbitrary",),
        ),
        cost_estimate=pl.CostEstimate(
            flops=2 * B * F,
            transcendentals=0,
            bytes_accessed=B * F * 4 + F * n_pad * 4 + B * 4,
        ),
    )(b_padded, x3, wt_padded)

    return out.reshape(B, 1)[:n_rows]
